# baseline (speedup 1.0000x reference)
import jax
import jax.numpy as jnp
from jax.experimental import pallas as pl

N, D = 10000, 128
TILE = 1000


def _cell_body(x_ref, w0t_ref, b0_ref, s1_ref, c1_ref, s2_ref, c2_ref,
               w2t_ref, b2_ref, wfa_ref, wfb_ref, bfc_ref, o_ref):
    h1 = jnp.dot(x_ref[...], w0t_ref[...],
                 preferred_element_type=jnp.float32) + b0_ref[...]
    t1 = jnp.maximum(h1 * s1_ref[...] + c1_ref[...], 0.0)
    t2 = jnp.maximum(h1 * s2_ref[...] + c2_ref[...], 0.0)
    h3 = jnp.dot(t2, w2t_ref[...],
                 preferred_element_type=jnp.float32) + b2_ref[...]
    h3 = jnp.maximum(h3, 0.0)
    o_ref[...] = (jnp.dot(t1, wfa_ref[...], preferred_element_type=jnp.float32)
                  + jnp.dot(h3, wfb_ref[...], preferred_element_type=jnp.float32)
                  + bfc_ref[...])


def kernel(x, edge_index, W0, b0, W2, b2, bn1_g, bn1_b, bn2_g, bn2_b, Wfc, bfc):
    del edge_index
    inv_std = 1.0 / jnp.sqrt(jnp.float32(1.0 + 1e-5))
    w0t = W0.T
    w2t = W2.T
    wfa = Wfc[:, :D].T
    wfb = Wfc[:, D:].T
    row = lambda v: v.reshape(1, D)
    s1 = row(bn1_g * inv_std)
    s2 = row(bn2_g * inv_std)

    grid = (N // TILE,)
    full = lambda shape: pl.BlockSpec(shape, lambda i: (0,) * len(shape))
    out = pl.pallas_call(
        _cell_body,
        grid=grid,
        in_specs=[
            pl.BlockSpec((TILE, D), lambda i: (i, 0)),
            full((D, D)),
            full((1, D)),
            full((1, D)),
            full((1, D)),
            full((1, D)),
            full((1, D)),
            full((D, D)),
            full((1, D)),
            full((D, D)),
            full((D, D)),
            full((1, D)),
        ],
        out_specs=pl.BlockSpec((TILE, D), lambda i: (i, 0)),
        out_shape=jax.ShapeDtypeStruct((N, D), jnp.float32),
    )(x, w0t, row(b0), s1, row(bn1_b), s2, row(bn2_b),
      w2t, row(b2), wfa, wfb, row(bfc))
    return out


# speedup vs baseline: 1.0600x; 1.0600x over previous
import jax
import jax.numpy as jnp
from jax.experimental import pallas as pl

N, D = 10000, 128
TILE = 1000


def _cell_body(x_ref, w_ref, v_ref, o_ref):
    bf = jnp.bfloat16
    f32 = jnp.float32
    xb = x_ref[...].astype(bf)
    h1 = jnp.dot(xb, w_ref[0], preferred_element_type=f32) + v_ref[0:1]
    t1 = jnp.maximum(h1 * v_ref[1:2] + v_ref[2:3], 0.0)
    t2 = jnp.maximum(h1 * v_ref[3:4] + v_ref[4:5], 0.0)
    h3 = jnp.dot(t2.astype(bf), w_ref[1], preferred_element_type=f32)
    h3 = jnp.maximum(h3 + v_ref[5:6], 0.0)
    o_ref[...] = (jnp.dot(t1.astype(bf), w_ref[2], preferred_element_type=f32)
                  + jnp.dot(h3.astype(bf), w_ref[3], preferred_element_type=f32)
                  + v_ref[6:7])


def kernel(x, edge_index, W0, b0, W2, b2, bn1_g, bn1_b, bn2_g, bn2_b, Wfc, bfc):
    del edge_index
    inv_std = 1.0 / jnp.sqrt(jnp.float32(1.0 + 1e-5))
    wpack = jnp.stack([W0.T, W2.T, Wfc[:, :D].T, Wfc[:, D:].T]
                      ).astype(jnp.bfloat16)
    vpack = jnp.stack([b0, bn1_g * inv_std, bn1_b, bn2_g * inv_std,
                       bn2_b, b2, bfc, jnp.zeros_like(b0)])

    grid = (N // TILE,)
    out = pl.pallas_call(
        _cell_body,
        grid=grid,
        in_specs=[
            pl.BlockSpec((TILE, D), lambda i: (i, 0)),
            pl.BlockSpec((4, D, D), lambda i: (0, 0, 0)),
            pl.BlockSpec((8, D), lambda i: (0, 0)),
        ],
        out_specs=pl.BlockSpec((TILE, D), lambda i: (i, 0)),
        out_shape=jax.ShapeDtypeStruct((N, D), jnp.float32),
    )(x, wpack, vpack)
    return out


# speedup vs baseline: 1.5147x; 1.4290x over previous
import jax
import jax.numpy as jnp
from jax.experimental import pallas as pl
from jax.experimental.pallas import tpu as pltpu

N, D = 10000, 128
TILE = 1000

_TN = (((1,), (1,)), ((), ()))


def _cell_body(x_ref, w0_ref, w2_ref, wfc_ref, b0_ref, g1_ref, c1_ref,
               g2_ref, c2_ref, b2_ref, bfc_ref, o_ref):
    bf = jnp.bfloat16
    f32 = jnp.float32
    inv_std = jnp.float32(0.9999950000374997)
    b0 = b0_ref[...]
    s1 = g1_ref[...] * inv_std
    s2 = g2_ref[...] * inv_std
    c1 = b0 * s1 + c1_ref[...]
    c2 = b0 * s2 + c2_ref[...]

    xb = x_ref[...].astype(bf)
    h1 = jax.lax.dot_general(xb, w0_ref[...].astype(bf), _TN,
                             preferred_element_type=f32)
    t1 = jnp.maximum(h1 * s1 + c1, 0.0)
    t2 = jnp.maximum(h1 * s2 + c2, 0.0)
    h3 = jax.lax.dot_general(t2.astype(bf), w2_ref[...].astype(bf), _TN,
                             preferred_element_type=f32)
    h3 = jnp.maximum(h3 + b2_ref[...], 0.0)
    wfc = wfc_ref[...].astype(bf)
    o_ref[...] = (jax.lax.dot_general(t1.astype(bf), wfc[:, :D], _TN,
                                      preferred_element_type=f32)
                  + jax.lax.dot_general(h3.astype(bf), wfc[:, D:], _TN,
                                        preferred_element_type=f32)
                  + bfc_ref[...])


def kernel(x, edge_index, W0, b0, W2, b2, bn1_g, bn1_b, bn2_g, bn2_b, Wfc, bfc):
    del edge_index
    row = lambda v: v.reshape(1, D)
    grid = (N // TILE,)
    vec = pl.BlockSpec((1, D), lambda i: (0, 0))
    out = pl.pallas_call(
        _cell_body,
        grid=grid,
        in_specs=[
            pl.BlockSpec((TILE, D), lambda i: (i, 0)),
            pl.BlockSpec((D, D), lambda i: (0, 0)),
            pl.BlockSpec((D, D), lambda i: (0, 0)),
            pl.BlockSpec((D, 2 * D), lambda i: (0, 0)),
            vec, vec, vec, vec, vec, vec, vec,
        ],
        out_specs=pl.BlockSpec((TILE, D), lambda i: (i, 0)),
        out_shape=jax.ShapeDtypeStruct((N, D), jnp.float32),
        compiler_params=pltpu.CompilerParams(
            dimension_semantics=("parallel",)),
    )(x, W0, W2, Wfc, row(b0), row(bn1_g), row(bn1_b), row(bn2_g),
      row(bn2_b), row(b2), row(bfc))
    return out
